# baseline (speedup 1.0000x reference)
import functools

import jax
import jax.numpy as jnp
from jax import lax
from jax.experimental import pallas as pl
from jax.experimental.pallas import tpu as pltpu
from jax.experimental.pallas import tpu_sc as plsc

NC = 2
NS = 16
LANES = 128


def _chunking(K: int):
    C = 1
    return C, K // C, C * LANES


@functools.lru_cache(maxsize=None)
def _make_seg_sum(K: int, D: int, SEG: int):
    RPT = SEG // NS
    C, KC, CH = _chunking(K)
    NB = max(1, min(8, KC, (256 * 1024) // (CH * D * 4)))
    R_FULL = KC // NB
    REM = KC - R_FULL * NB
    mesh = plsc.VectorSubcoreMesh(
        core_axis_name="c", subcore_axis_name="s", num_cores=NC, num_subcores=NS
    )

    @functools.partial(
        pl.kernel,
        out_type=jax.ShapeDtypeStruct((NC, SEG, D), jnp.float32),
        mesh=mesh,
        compiler_params=pltpu.CompilerParams(use_tc_tiling_on_sc=False),
        scratch_types=[
            pltpu.VMEM((K, LANES), jnp.int32),
            pltpu.VMEM((KC, CH), jnp.int32),
            pltpu.VMEM((NB, C * LANES, D), jnp.float32),
            pltpu.VMEM_SHARED((SEG, D), jnp.float32),
            pltpu.SemaphoreType.DMA((NB,)),
            pltpu.SemaphoreType.DMA((NB,)),
        ],
    )
    def seg_sum(table, src3, dst3, zeros, out, src_v, dst_v, rows_v, acc_sh,
                gsem, ssem):
        cid = lax.axis_index("c")
        sid = lax.axis_index("s")
        pltpu.sync_copy(zeros, acc_sh.at[pl.ds(sid * RPT, RPT)])
        pltpu.sync_copy(src3.at[cid, sid], src_v)
        pltpu.sync_copy(dst3.at[cid, sid], dst_v)
        plsc.subcore_barrier()

        def gather(j, b):
            return pltpu.async_copy(
                table.at[src_v.at[j]], rows_v.at[b], gsem.at[b]
            )

        for b in range(NB):
            gather(b, b)

        def round_body(r, carry):
            base = r * NB
            scat = []
            for b in range(NB):
                pltpu.make_async_copy(
                    table.at[src_v.at[base + b]], rows_v.at[b], gsem.at[b],
                ).wait()
                scat.append(
                    pltpu.async_copy(
                        rows_v.at[b], acc_sh.at[dst_v.at[base + b]],
                        ssem.at[b], add=True,
                    )
                )
            for b in range(NB):
                scat[b].wait()
                nxt = base + NB + b

                @pl.when(nxt < KC)
                def _():
                    gather(nxt, b)

            return carry

        lax.fori_loop(0, R_FULL, round_body, 0, unroll=False)

        for b in range(REM):
            j = R_FULL * NB + b
            pltpu.make_async_copy(
                table.at[src_v.at[j]], rows_v.at[b], gsem.at[b]
            ).wait()
            pltpu.sync_copy(
                rows_v.at[b], acc_sh.at[dst_v.at[j]], add=True
            )

        plsc.subcore_barrier()
        pltpu.sync_copy(
            acc_sh.at[pl.ds(sid * RPT, RPT)], out.at[cid, pl.ds(sid * RPT, RPT)]
        )

    return seg_sum


def _pad_indices(idx, count, pad_base, K):
    total = NC * NS * K * LANES
    pad_len = total - count
    fill = pad_base + (jnp.arange(pad_len, dtype=jnp.int32) % LANES)
    idx = jnp.concatenate([idx, fill])
    return idx.reshape(NC, NS, K * LANES)


def _seg_sum_partials(table, src3, dst3, K, SEG):
    D = table.shape[1]
    _, KC, CH = _chunking(K)
    zeros = jnp.zeros((SEG // NS, D), dtype=jnp.float32)
    return _make_seg_sum(K, D, SEG)(
        table, src3.reshape(NC, NS, K, LANES), dst3.reshape(NC, NS, KC, CH), zeros
    )


def _proj_body(x_ref, w_ref, o_ref):
    o_ref[...] = jnp.dot(x_ref[...], w_ref[...], preferred_element_type=jnp.float32)


def _mlp_body(n_rows, h_ref, p_ref, wa_ref, ba_ref, wb_ref, bb_ref, g_ref, bt_ref, o_ref):
    agg = p_ref[0, :n_rows, :] + p_ref[1, :n_rows, :]
    z = jnp.maximum(h_ref[...] + agg + ba_ref[...], 0.0)
    t = jnp.dot(z, wb_ref[...], preferred_element_type=jnp.float32) + bb_ref[...]
    o_ref[...] = jnp.maximum(t * g_ref[...] + bt_ref[...], 0.0)


def _mlp2_body(n_rows, h_ref, p_ref, wa_ref, ba_ref, wb_ref, bb_ref, g_ref, bt_ref, o_ref):
    agg = p_ref[0, :n_rows, :] + p_ref[1, :n_rows, :]
    z = jnp.maximum(
        jnp.dot(h_ref[...] + agg, wa_ref[...], preferred_element_type=jnp.float32)
        + ba_ref[...],
        0.0,
    )
    t = jnp.dot(z, wb_ref[...], preferred_element_type=jnp.float32) + bb_ref[...]
    o_ref[...] = jnp.maximum(t * g_ref[...] + bt_ref[...], 0.0)


def _head_body(n_rows, p_ref, w_ref, b_ref, o_ref):
    p = p_ref[0, :n_rows, :] + p_ref[1, :n_rows, :]
    o_ref[...] = jnp.dot(p, w_ref[...], preferred_element_type=jnp.float32) + b_ref[...]


def _ceil_to(v, m):
    return -(-v // m) * m


def kernel(x, edge_index, batch, W1, b1, W2, b2, bn1_g, bn1_b,
           W3, b3, W4, b4, bn2_g, bn2_b, W5, b5):
    N, D = x.shape
    E = edge_index.shape[1]
    G = 512

    f32 = jnp.float32
    src = edge_index[0]
    dst = edge_index[1]

    K_e = -(-E // (NC * NS * LANES))
    SEG_n = _ceil_to(N + LANES, NS * 8)
    src3 = _pad_indices(src, E, 0, K_e)
    dst3 = _pad_indices(dst, E, N, K_e)

    K_p = -(-N // (NC * NS * LANES))
    SEG_g = _ceil_to(G + LANES, NS * 8)
    iota3 = _pad_indices(jnp.arange(N, dtype=jnp.int32), N, 0, K_p)
    batch3 = _pad_indices(batch, N, G, K_p)

    s1 = (bn1_g / jnp.sqrt(1.0 + 1e-5)).reshape(1, -1)
    s2 = (bn2_g / jnp.sqrt(1.0 + 1e-5)).reshape(1, -1)
    b1r, b2r = b1.reshape(1, -1), b2.reshape(1, -1)
    b3r, b4r = b3.reshape(1, -1), b4.reshape(1, -1)
    bt1, bt2 = bn1_b.reshape(1, -1), bn2_b.reshape(1, -1)
    b5r = b5.reshape(1, -1)

    H1 = W1.shape[1]
    H2 = W4.shape[1]

    y1 = pl.pallas_call(
        _proj_body, out_shape=jax.ShapeDtypeStruct((N, H1), f32)
    )(x, W1)

    p1 = _seg_sum_partials(y1, src3, dst3, K_e, SEG_n)

    h1 = pl.pallas_call(
        functools.partial(_mlp_body, N), out_shape=jax.ShapeDtypeStruct((N, H1), f32)
    )(y1, p1, W1, b1r, W2, b2r, s1, bt1)

    p2 = _seg_sum_partials(h1, src3, dst3, K_e, SEG_n)

    h2 = pl.pallas_call(
        functools.partial(_mlp2_body, N), out_shape=jax.ShapeDtypeStruct((N, H2), f32)
    )(h1, p2, W3, b3r, W4, b4r, s2, bt2)

    p3 = _seg_sum_partials(h2, iota3, batch3, K_p, SEG_g)

    out = pl.pallas_call(
        functools.partial(_head_body, G), out_shape=jax.ShapeDtypeStruct((G, 1), f32)
    )(p3, W5, b5r)

    return out[:, 0]


# speedup vs baseline: 1.0241x; 1.0241x over previous
import functools

import jax
import jax.numpy as jnp
from jax import lax
from jax.experimental import pallas as pl
from jax.experimental.pallas import tpu as pltpu
from jax.experimental.pallas import tpu_sc as plsc

NC = 2
NS = 16
LANES = 128


def _chunking(K: int):
    C = 1
    return C, K // C, C * LANES


@functools.lru_cache(maxsize=None)
def _make_seg_sum(K: int, D: int, SEG: int):
    RPT = SEG // NS
    C, KC, CH = _chunking(K)
    NB = max(1, min(8, KC, (256 * 1024) // (CH * D * 4)))
    R_FULL = KC // NB
    REM = KC - R_FULL * NB
    mesh = plsc.VectorSubcoreMesh(
        core_axis_name="c", subcore_axis_name="s", num_cores=NC, num_subcores=NS
    )

    @functools.partial(
        pl.kernel,
        out_type=jax.ShapeDtypeStruct((NC, SEG, D), jnp.float32),
        mesh=mesh,
        compiler_params=pltpu.CompilerParams(use_tc_tiling_on_sc=False),
        scratch_types=[
            pltpu.VMEM((K, LANES), jnp.int32),
            pltpu.VMEM((KC, CH), jnp.int32),
            pltpu.VMEM((NB, C * LANES, D), jnp.float32),
            pltpu.VMEM_SHARED((SEG, D), jnp.float32),
            pltpu.SemaphoreType.DMA((NB,)),
            pltpu.SemaphoreType.DMA((NB,)),
        ],
    )
    def seg_sum(table, src3, dst3, zeros, out, src_v, dst_v, rows_v, acc_sh,
                gsem, ssem):
        cid = lax.axis_index("c")
        sid = lax.axis_index("s")
        pltpu.sync_copy(zeros, acc_sh.at[pl.ds(sid * RPT, RPT)])
        pltpu.sync_copy(src3.at[cid, sid], src_v)
        pltpu.sync_copy(dst3.at[cid, sid], dst_v)
        plsc.subcore_barrier()

        def gather(j, b):
            return pltpu.async_copy(
                table.at[src_v.at[j]], rows_v.at[b], gsem.at[b]
            )

        for b in range(NB):
            gather(b, b)

        def round_body(r, carry):
            base = r * NB
            for b in range(NB):
                pltpu.make_async_copy(
                    table.at[src_v.at[base + b]], rows_v.at[b], gsem.at[b],
                ).wait()
                pltpu.sync_copy(
                    rows_v.at[b], acc_sh.at[dst_v.at[base + b]], add=True
                )
                nxt = base + NB + b

                @pl.when(nxt < KC)
                def _():
                    gather(nxt, b)

            return carry

        lax.fori_loop(0, R_FULL, round_body, 0, unroll=False)

        for b in range(REM):
            j = R_FULL * NB + b
            pltpu.make_async_copy(
                table.at[src_v.at[j]], rows_v.at[b], gsem.at[b]
            ).wait()
            pltpu.sync_copy(
                rows_v.at[b], acc_sh.at[dst_v.at[j]], add=True
            )

        plsc.subcore_barrier()
        pltpu.sync_copy(
            acc_sh.at[pl.ds(sid * RPT, RPT)], out.at[cid, pl.ds(sid * RPT, RPT)]
        )

    return seg_sum


def _pad_indices(idx, count, pad_base, K):
    total = NC * NS * K * LANES
    pad_len = total - count
    fill = pad_base + (jnp.arange(pad_len, dtype=jnp.int32) % LANES)
    idx = jnp.concatenate([idx, fill])
    return idx.reshape(NC, NS, K * LANES)


def _seg_sum_partials(table, src3, dst3, K, SEG):
    D = table.shape[1]
    _, KC, CH = _chunking(K)
    zeros = jnp.zeros((SEG // NS, D), dtype=jnp.float32)
    return _make_seg_sum(K, D, SEG)(
        table, src3.reshape(NC, NS, K, LANES), dst3.reshape(NC, NS, KC, CH), zeros
    )


def _proj_body(x_ref, w_ref, o_ref):
    o_ref[...] = jnp.dot(x_ref[...], w_ref[...], preferred_element_type=jnp.float32)


def _mlp_body(n_rows, h_ref, p_ref, wa_ref, ba_ref, wb_ref, bb_ref, g_ref, bt_ref, o_ref):
    agg = p_ref[0, :n_rows, :] + p_ref[1, :n_rows, :]
    z = jnp.maximum(h_ref[...] + agg + ba_ref[...], 0.0)
    t = jnp.dot(z, wb_ref[...], preferred_element_type=jnp.float32) + bb_ref[...]
    o_ref[...] = jnp.maximum(t * g_ref[...] + bt_ref[...], 0.0)


def _mlp2_body(n_rows, h_ref, p_ref, wa_ref, ba_ref, wb_ref, bb_ref, g_ref, bt_ref, o_ref):
    agg = p_ref[0, :n_rows, :] + p_ref[1, :n_rows, :]
    z = jnp.maximum(
        jnp.dot(h_ref[...] + agg, wa_ref[...], preferred_element_type=jnp.float32)
        + ba_ref[...],
        0.0,
    )
    t = jnp.dot(z, wb_ref[...], preferred_element_type=jnp.float32) + bb_ref[...]
    o_ref[...] = jnp.maximum(t * g_ref[...] + bt_ref[...], 0.0)


def _head_body(n_rows, p_ref, w_ref, b_ref, o_ref):
    p = p_ref[0, :n_rows, :] + p_ref[1, :n_rows, :]
    o_ref[...] = jnp.dot(p, w_ref[...], preferred_element_type=jnp.float32) + b_ref[...]


def _ceil_to(v, m):
    return -(-v // m) * m


def kernel(x, edge_index, batch, W1, b1, W2, b2, bn1_g, bn1_b,
           W3, b3, W4, b4, bn2_g, bn2_b, W5, b5):
    N, D = x.shape
    E = edge_index.shape[1]
    G = 512

    f32 = jnp.float32
    src = edge_index[0]
    dst = edge_index[1]

    K_e = -(-E // (NC * NS * LANES))
    SEG_n = _ceil_to(N + LANES, NS * 8)
    src3 = _pad_indices(src, E, 0, K_e)
    dst3 = _pad_indices(dst, E, N, K_e)

    K_p = -(-N // (NC * NS * LANES))
    SEG_g = _ceil_to(G + LANES, NS * 8)
    iota3 = _pad_indices(jnp.arange(N, dtype=jnp.int32), N, 0, K_p)
    batch3 = _pad_indices(batch, N, G, K_p)

    s1 = (bn1_g / jnp.sqrt(1.0 + 1e-5)).reshape(1, -1)
    s2 = (bn2_g / jnp.sqrt(1.0 + 1e-5)).reshape(1, -1)
    b1r, b2r = b1.reshape(1, -1), b2.reshape(1, -1)
    b3r, b4r = b3.reshape(1, -1), b4.reshape(1, -1)
    bt1, bt2 = bn1_b.reshape(1, -1), bn2_b.reshape(1, -1)
    b5r = b5.reshape(1, -1)

    H1 = W1.shape[1]
    H2 = W4.shape[1]

    y1 = pl.pallas_call(
        _proj_body, out_shape=jax.ShapeDtypeStruct((N, H1), f32)
    )(x, W1)

    p1 = _seg_sum_partials(y1, src3, dst3, K_e, SEG_n)

    h1 = pl.pallas_call(
        functools.partial(_mlp_body, N), out_shape=jax.ShapeDtypeStruct((N, H1), f32)
    )(y1, p1, W1, b1r, W2, b2r, s1, bt1)

    p2 = _seg_sum_partials(h1, src3, dst3, K_e, SEG_n)

    h2 = pl.pallas_call(
        functools.partial(_mlp2_body, N), out_shape=jax.ShapeDtypeStruct((N, H2), f32)
    )(h1, p2, W3, b3r, W4, b4r, s2, bt2)

    p3 = _seg_sum_partials(h2, iota3, batch3, K_p, SEG_g)

    out = pl.pallas_call(
        functools.partial(_head_body, G), out_shape=jax.ShapeDtypeStruct((G, 1), f32)
    )(p3, W5, b5r)

    return out[:, 0]


# speedup vs baseline: 1.1147x; 1.0884x over previous
import functools

import jax
import jax.numpy as jnp
from jax import lax
from jax.experimental import pallas as pl
from jax.experimental.pallas import tpu as pltpu
from jax.experimental.pallas import tpu_sc as plsc

NC = 2
NS = 16
LANES = 128


def _chunking(K: int):
    C = 1
    return C, K // C, C * LANES


@functools.lru_cache(maxsize=None)
def _make_seg_sum(K: int, D: int, SEG: int):
    RPT = SEG // NS
    C, KC, CH = _chunking(K)
    NB = max(1, min(8, KC, (256 * 1024) // (CH * D * 4)))
    R_FULL = KC // NB
    REM = KC - R_FULL * NB
    mesh = plsc.VectorSubcoreMesh(
        core_axis_name="c", subcore_axis_name="s", num_cores=NC, num_subcores=NS
    )

    @functools.partial(
        pl.kernel,
        out_type=jax.ShapeDtypeStruct((NC, SEG, D), jnp.float32),
        mesh=mesh,
        compiler_params=pltpu.CompilerParams(use_tc_tiling_on_sc=False),
        scratch_types=[
            pltpu.VMEM((K, LANES), jnp.int32),
            pltpu.VMEM((KC, CH), jnp.int32),
            pltpu.VMEM((NB, C * LANES, D), jnp.float32),
            pltpu.VMEM_SHARED((SEG, D), jnp.float32),
            pltpu.SemaphoreType.DMA((NB,)),
            pltpu.SemaphoreType.DMA((NB,)),
        ],
    )
    def seg_sum(table, src3, dst3, zeros, out, src_v, dst_v, rows_v, acc_sh,
                gsem, ssem):
        cid = lax.axis_index("c")
        sid = lax.axis_index("s")
        pltpu.sync_copy(zeros, acc_sh.at[pl.ds(sid * RPT, RPT)])
        pltpu.sync_copy(src3.at[cid, sid], src_v)
        pltpu.sync_copy(dst3.at[cid, sid], dst_v)
        plsc.subcore_barrier()

        def gather(j, b):
            return pltpu.async_copy(
                table.at[src_v.at[j]], rows_v.at[b], gsem.at[b]
            )

        for b in range(NB):
            gather(b, b)

        def round_body(r, carry):
            base = r * NB
            for b in range(NB):
                pltpu.make_async_copy(
                    table.at[src_v.at[base + b]], rows_v.at[b], gsem.at[b],
                ).wait()
                pltpu.sync_copy(
                    rows_v.at[b], acc_sh.at[dst_v.at[base + b]], add=True
                )
                nxt = base + NB + b

                @pl.when(nxt < KC)
                def _():
                    gather(nxt, b)

            return carry

        lax.fori_loop(0, R_FULL, round_body, 0, unroll=False)

        for b in range(REM):
            j = R_FULL * NB + b
            pltpu.make_async_copy(
                table.at[src_v.at[j]], rows_v.at[b], gsem.at[b]
            ).wait()
            pltpu.sync_copy(
                rows_v.at[b], acc_sh.at[dst_v.at[j]], add=True
            )

        plsc.subcore_barrier()
        pltpu.sync_copy(
            acc_sh.at[pl.ds(sid * RPT, RPT)], out.at[cid, pl.ds(sid * RPT, RPT)]
        )

    return seg_sum


def _pad_indices(idx, count, pad_base, K):
    total = NC * NS * K * LANES
    pad_len = total - count
    fill = pad_base + (jnp.arange(pad_len, dtype=jnp.int32) % LANES)
    idx = jnp.concatenate([idx, fill])
    return idx.reshape(NC, NS, K * LANES)


def _seg_sum_partials(table, src3, dst3, K, SEG):
    D = table.shape[1]
    _, KC, CH = _chunking(K)
    zeros = jnp.zeros((SEG // NS, D), dtype=jnp.float32)
    return _make_seg_sum(K, D, SEG)(
        table, src3.reshape(NC, NS, K, LANES), dst3.reshape(NC, NS, KC, CH), zeros
    )


def _proj_body(x_ref, w_ref, o_ref):
    o_ref[...] = jnp.dot(x_ref[...], w_ref[...], preferred_element_type=jnp.float32)


def _mlp_body(n_rows, h_ref, p_ref, wa_ref, ba_ref, wb_ref, bb_ref, g_ref, bt_ref, o_ref):
    agg = p_ref[0, :n_rows, :] + p_ref[1, :n_rows, :]
    z = jnp.maximum(h_ref[...] + agg + ba_ref[...], 0.0)
    t = jnp.dot(z, wb_ref[...], preferred_element_type=jnp.float32) + bb_ref[...]
    o_ref[...] = jnp.maximum(t * g_ref[...] + bt_ref[...], 0.0)


def _mlp2_pool_head_body(n_rows, n_graphs, h_ref, p_ref, wa_ref, ba_ref,
                         wb_ref, bb_ref, g_ref, bt_ref, batch_ref, w5_ref,
                         b5_ref, o_ref):
    agg = p_ref[0, :n_rows, :] + p_ref[1, :n_rows, :]
    z = jnp.maximum(
        jnp.dot(h_ref[...] + agg, wa_ref[...], preferred_element_type=jnp.float32)
        + ba_ref[...],
        0.0,
    )
    t = jnp.dot(z, wb_ref[...], preferred_element_type=jnp.float32) + bb_ref[...]
    h2 = jnp.maximum(t * g_ref[...] + bt_ref[...], 0.0)
    gid = lax.broadcasted_iota(jnp.int32, (1, n_graphs), 1)
    oneh = (batch_ref[...] == gid).astype(jnp.float32)
    pooled = lax.dot_general(
        oneh, h2, (((0,), (0,)), ((), ())),
        preferred_element_type=jnp.float32,
    )
    o_ref[...] = (
        jnp.dot(pooled, w5_ref[...], preferred_element_type=jnp.float32)
        + b5_ref[...]
    )


def _ceil_to(v, m):
    return -(-v // m) * m


def kernel(x, edge_index, batch, W1, b1, W2, b2, bn1_g, bn1_b,
           W3, b3, W4, b4, bn2_g, bn2_b, W5, b5):
    N, D = x.shape
    E = edge_index.shape[1]
    G = 512

    f32 = jnp.float32
    src = edge_index[0]
    dst = edge_index[1]

    K_e = -(-E // (NC * NS * LANES))
    SEG_n = _ceil_to(N + LANES, NS * 8)
    src3 = _pad_indices(src, E, 0, K_e)
    dst3 = _pad_indices(dst, E, N, K_e)

    s1 = (bn1_g / jnp.sqrt(1.0 + 1e-5)).reshape(1, -1)
    s2 = (bn2_g / jnp.sqrt(1.0 + 1e-5)).reshape(1, -1)
    b1r, b2r = b1.reshape(1, -1), b2.reshape(1, -1)
    b3r, b4r = b3.reshape(1, -1), b4.reshape(1, -1)
    bt1, bt2 = bn1_b.reshape(1, -1), bn2_b.reshape(1, -1)
    b5r = b5.reshape(1, -1)

    H1 = W1.shape[1]
    H2 = W4.shape[1]

    y1 = pl.pallas_call(
        _proj_body, out_shape=jax.ShapeDtypeStruct((N, H1), f32)
    )(x, W1)

    p1 = _seg_sum_partials(y1, src3, dst3, K_e, SEG_n)

    h1 = pl.pallas_call(
        functools.partial(_mlp_body, N), out_shape=jax.ShapeDtypeStruct((N, H1), f32)
    )(y1, p1, W1, b1r, W2, b2r, s1, bt1)

    p2 = _seg_sum_partials(h1, src3, dst3, K_e, SEG_n)

    out = pl.pallas_call(
        functools.partial(_mlp2_pool_head_body, N, G),
        out_shape=jax.ShapeDtypeStruct((G, 1), f32),
    )(h1, p2, W3, b3r, W4, b4r, s2, bt2, batch.reshape(N, 1), W5, b5r)

    return out[:, 0]
